# Initial kernel scaffold; baseline (speedup 1.0000x reference)
import jax
import jax.numpy as jnp
from jax.experimental import pallas as pl


def kernel(x, edge_index, edge_weight, noisy_value, W1, b1, W2, b2, Wf1, bf1, Wf2, bf2, Wf3, bf3):
    raise NotImplementedError("write your pallas kernel here")


# speedup vs baseline: 1.1799x; 1.1799x over previous
import jax
import jax.numpy as jnp
from jax import lax
from jax.experimental import pallas as pl
from jax.experimental.pallas import tpu as pltpu
from jax.experimental.pallas import tpu_sc as plsc

N = 10
E = 64
F = 4096
H = 64
NP = 16
L = 16
C = H // L

ROWS0 = 16
ROWS = 272


def _rsqrt(v):
    i = plsc.bitcast(v, jnp.int32)
    i = 0x5F3759DF - lax.shift_right_logical(i, 1)
    y = plsc.bitcast(i, jnp.float32)
    for _ in range(3):
        y = y * (1.5 - 0.5 * v * y * y)
    return y


def _bf16rne(v):
    i = plsc.bitcast(v, jnp.int32)
    lsb = jnp.bitwise_and(lax.shift_right_logical(i, 16), 1)
    r = jnp.bitwise_and(i + 0x7FFF + lsb, -65536)
    return plsc.bitcast(r, jnp.float32)


def _onehot(iota, idx):
    return (1 - jnp.minimum(jnp.abs(iota - idx), 1)).astype(jnp.float32)


def _body(x_h, eix_h, ew_h, nv_h, w1_h, consts_h, w2_h,
          wf1_h, wf1g_h, bf1_h, wf2_h, bf2_h, wf3_h,
          out_h,
          nv_v, w_v, acc_v, part_v, shr,
          x_v, eix_v, ew_v, w1_v, b1_v, w2_v, consts_v,
          wf1g_v, bf1_v, wf2_v, bf2_v, wf3_v,
          dinv_v, m_v, xw_v, h_v, res_v):
    core = lax.axis_index("c")
    wid = lax.axis_index("s")
    zero16 = jnp.zeros((L,), jnp.float32)
    iota = lax.broadcasted_iota(jnp.int32, (L,), 0)

    @pl.when(jnp.logical_and(core == 0, wid == 0))
    def _():
        pltpu.sync_copy(x_h, x_v)
        pltpu.sync_copy(eix_h, eix_v)
        pltpu.sync_copy(ew_h, ew_v)
        pltpu.sync_copy(w1_h, w1_v)
        pltpu.sync_copy(consts_h, consts_v)
        pltpu.sync_copy(w2_h, w2_v)
        pltpu.sync_copy(wf1g_h, wf1g_v)
        pltpu.sync_copy(bf1_h, bf1_v)
        pltpu.sync_copy(wf2_h, wf2_v)
        pltpu.sync_copy(bf2_h, bf2_v)
        pltpu.sync_copy(wf3_h, wf3_v)
        pltpu.sync_copy(nv_h.at[pl.ds(0, ROWS0)], nv_v.at[pl.ds(0, ROWS0)])
        pltpu.sync_copy(wf1_h.at[pl.ds(N * H, ROWS0 * H)],
                        w_v.at[pl.ds(0, ROWS0 * H)])

        mask = iota < N
        deg = jnp.where(mask, 1.0, 0.0)
        for g in range(E // L):
            d16 = eix_v[pl.ds(E + g * L, L)]
            wv = ew_v[pl.ds(g * L, L)]
            for t in range(L):
                deg = deg + wv[t] * _onehot(iota, d16[t])
        dinv = jnp.where(mask, _rsqrt(jnp.where(mask, deg, 1.0)), 0.0)
        dinv_v[...] = dinv

        dv2 = dinv * dinv
        for r in range(NP):
            m_v[pl.ds(r * NP, NP)] = dv2[r] * _onehot(iota, r)
        for g in range(E // L):
            s16 = eix_v[pl.ds(g * L, L)]
            d16 = eix_v[pl.ds(E + g * L, L)]
            wv = ew_v[pl.ds(g * L, L)]
            dvs = plsc.load_gather(dinv_v, [s16])
            dvd = plsc.load_gather(dinv_v, [d16])
            nrm = dvs * wv * dvd
            for t in range(L):
                off = d16[t] * NP
                row = m_v[pl.ds(off, NP)]
                m_v[pl.ds(off, NP)] = row + nrm[t] * _onehot(iota, s16[t])

        xrows = [x_v[pl.ds(n * NP, L)] for n in range(N)]
        for c in range(C):
            w1c = [w1_v[pl.ds(k * H + c * L, L)] for k in range(N)]
            for n in range(N):
                a = xrows[n][0] * w1c[0]
                for k in range(1, N):
                    a = a + xrows[n][k] * w1c[k]
                xw_v[pl.ds(n * H + c * L, L)] = a

        mrows = [m_v[pl.ds(i * NP, NP)] for i in range(N)]
        for c in range(C):
            xwc = [xw_v[pl.ds(k * H + c * L, L)] for k in range(N)]
            b1c = b1_v[pl.ds(c * L, L)]
            for i in range(N):
                a = b1c
                for k in range(N):
                    a = a + mrows[i][k] * xwc[k]
                h_v[pl.ds(i * H + c * L, L)] = _bf16rne(jnp.maximum(a, 0.0))

        w2c = [w2_v[pl.ds(c * L, L)] for c in range(C)]
        xw2 = zero16
        for n in range(N):
            a = h_v[pl.ds(n * H, L)] * w2c[0]
            for c in range(1, C):
                a = a + h_v[pl.ds(n * H + c * L, L)] * w2c[c]
            xw2 = jnp.where(iota == n, jnp.sum(a), xw2)
        consts = consts_v[...]
        b2s = consts[0]
        gvec = zero16
        for i in range(N):
            gi = jnp.sum(mrows[i] * xw2) + b2s
            gvec = jnp.where(iota == i, gi, gvec)
        gvec = _bf16rne(gvec)

        facc = [bf1_v[pl.ds(c * L, L)] for c in range(C)]
        for i in range(N):
            for c in range(C):
                facc[c] = facc[c] + gvec[i] * wf1g_v[pl.ds(i * H + c * L, L)]

        nvv = nv_v[pl.ds(0, L)]
        for t in range(ROWS0):
            nvt = nvv[t]
            for c in range(C):
                facc[c] = facc[c] + nvt * w_v[pl.ds(t * H + c * L, L)]
        for c in range(C):
            acc_v[pl.ds(c * L, L)] = facc[c]
        pltpu.sync_copy(acc_v, shr.at[pl.ds(0, H)])

    @pl.when(jnp.logical_and(core == 0, wid > 0))
    def _():
        base = ROWS0 + (wid - 1) * ROWS
        pltpu.sync_copy(nv_h.at[pl.ds(base, ROWS)], nv_v)
        pltpu.sync_copy(wf1_h.at[pl.ds((N + base) * H, ROWS * H)], w_v)

        def mv(b, accs):
            accs = list(accs)
            nvv = nv_v[pl.ds(b * L, L)]
            for t in range(L):
                nvt = nvv[t]
                for c in range(C):
                    accs[c] = accs[c] + nvt * w_v[pl.ds(b * (L * H) + t * H
                                                       + c * L, L)]
            return tuple(accs)

        accs = lax.fori_loop(0, ROWS // L, mv, (zero16,) * C)
        for c in range(C):
            acc_v[pl.ds(c * L, L)] = accs[c]
        pltpu.sync_copy(acc_v, shr.at[pl.ds(wid * H, H)])

    @pl.when(core == 0)
    def _():
        plsc.subcore_barrier()

    @pl.when(jnp.logical_and(core == 0, wid == 0))
    def _():
        pltpu.sync_copy(shr, part_v)
        tot = [part_v[pl.ds(c * L, L)] for c in range(C)]
        for r in range(1, NP):
            for c in range(C):
                tot[c] = tot[c] + part_v[pl.ds(r * H + c * L, L)]
        h1 = [_bf16rne(jnp.maximum(tot[c], 0.0)) for c in range(C)]

        acc2 = [bf2_v[pl.ds(c * L, L)] for c in range(C)]
        for k in range(H):
            hk = h1[k // L][k % L]
            for c in range(C):
                acc2[c] = acc2[c] + hk * wf2_v[pl.ds(k * H + c * L, L)]
        out = consts_v[...][1]
        for c in range(C):
            h2c = _bf16rne(jnp.maximum(acc2[c], 0.0))
            out = out + jnp.sum(h2c * wf3_v[pl.ds(c * L, L)])
        res_v[...] = jnp.full((L,), out, jnp.float32)
        pltpu.sync_copy(res_v, out_h)


def _make_sc_call():
    mesh = plsc.VectorSubcoreMesh(core_axis_name="c", subcore_axis_name="s")
    scratch = [
        pltpu.VMEM((ROWS,), jnp.float32),
        pltpu.VMEM((ROWS * H,), jnp.float32),
        pltpu.VMEM((H,), jnp.float32),
        pltpu.VMEM((NP * H,), jnp.float32),
        pltpu.VMEM_SHARED((NP * H,), jnp.float32),
        pltpu.VMEM((NP * NP,), jnp.float32),
        pltpu.VMEM((2 * E,), jnp.int32),
        pltpu.VMEM((E,), jnp.float32),
        pltpu.VMEM((NP * H,), jnp.float32),
        pltpu.VMEM((H,), jnp.float32),
        pltpu.VMEM((H,), jnp.float32),
        pltpu.VMEM((L,), jnp.float32),
        pltpu.VMEM((NP * H,), jnp.float32),
        pltpu.VMEM((H,), jnp.float32),
        pltpu.VMEM((H * H,), jnp.float32),
        pltpu.VMEM((H,), jnp.float32),
        pltpu.VMEM((H,), jnp.float32),
        pltpu.VMEM((NP,), jnp.float32),
        pltpu.VMEM((NP * NP,), jnp.float32),
        pltpu.VMEM((N * H,), jnp.float32),
        pltpu.VMEM((N * H,), jnp.float32),
        pltpu.VMEM((L,), jnp.float32),
    ]
    return pl.kernel(
        _body,
        out_type=jax.ShapeDtypeStruct((L,), jnp.float32),
        mesh=mesh,
        scratch_types=scratch,
        compiler_params=pltpu.CompilerParams(needs_layout_passes=False),
    )


def _rne(a):
    i = lax.bitcast_convert_type(a, jnp.int32)
    lsb = jnp.bitwise_and(lax.shift_right_logical(i, 16), 1)
    r = jnp.bitwise_and(i + 0x7FFF + lsb, jnp.int32(-65536))
    return lax.bitcast_convert_type(r, jnp.float32)


@jax.jit
def kernel(x, edge_index, edge_weight, noisy_value,
           W1, b1, W2, b2, Wf1, bf1, Wf2, bf2, Wf3, bf3):
    x = _rne(x)
    W1 = _rne(W1)
    Wf1 = _rne(Wf1)
    xp = jnp.zeros((NP, NP), jnp.float32).at[:N, :N].set(x).reshape(NP * NP)
    w1p = jnp.zeros((NP, H), jnp.float32).at[:N].set(W1).reshape(NP * H)
    wf1g = (jnp.zeros((NP, H), jnp.float32).at[:N].set(Wf1[:N])
            .reshape(NP * H))
    nv = _rne(noisy_value.reshape(F))
    w2 = _rne(W2.reshape(H))
    wf3 = _rne(Wf3.reshape(H))
    eix = edge_index.astype(jnp.int32).reshape(2 * E)
    consts = jnp.zeros((L,), jnp.float32).at[0].set(b2[0]).at[1].set(bf3[0])
    out16 = _make_sc_call()(
        xp, eix, edge_weight, nv, w1p, consts, w2,
        Wf1.reshape((N + F) * H), wf1g, bf1, _rne(Wf2).reshape(H * H),
        bf2, wf3)
    return out16[:1].reshape(1, 1)
